# baseline (speedup 1.0000x reference)
import functools

import jax
import jax.numpy as jnp
from jax import lax
from jax.experimental import pallas as pl
from jax.experimental.pallas import tpu as pltpu
from jax.experimental.pallas import tpu_sc as plsc

B = 16384
F = 64
NC = 2
NS = 16
L = 16
NW = NC * NS
BPW = B // NW
GROUPS = BPW // L

_mesh = plsc.VectorSubcoreMesh(core_axis_name="c", subcore_axis_name="s")


@functools.partial(
    pl.kernel,
    mesh=_mesh,
    out_type=jax.ShapeDtypeStruct((B,), jnp.float32),
    compiler_params=pltpu.CompilerParams(
        needs_layout_passes=False, use_tc_tiling_on_sc=False),
    scratch_types=[
        pltpu.VMEM((BPW,), jnp.int32),
        pltpu.VMEM((BPW,), jnp.int32),
        pltpu.VMEM((BPW, F), jnp.float32),
        pltpu.VMEM((BPW, F), jnp.float32),
        pltpu.VMEM((BPW,), jnp.float32),
        pltpu.VMEM((BPW,), jnp.float32),
        pltpu.VMEM((BPW,), jnp.float32),
        pltpu.SemaphoreType.DMA,
        pltpu.SemaphoreType.DMA,
        pltpu.SemaphoreType.DMA,
        pltpu.SemaphoreType.DMA,
    ],
)
def _sc_gather_dot(users_hbm, items_hbm, ue_hbm, ie_hbm, ub_hbm, ib_hbm,
                   out_hbm, uidx_v, iidx_v, urows_v, irows_v, ubias_v,
                   ibias_v, res_v, sem_u, sem_i, sem_ub, sem_ib):
    wid = lax.axis_index("s") * NC + lax.axis_index("c")
    base = wid * BPW

    pltpu.sync_copy(users_hbm.at[pl.ds(base, BPW)], uidx_v)
    pltpu.sync_copy(items_hbm.at[pl.ds(base, BPW)], iidx_v)

    cu = pltpu.async_copy(ue_hbm.at[uidx_v], urows_v, sem_u)
    ci = pltpu.async_copy(ie_hbm.at[iidx_v], irows_v, sem_i)
    cub = pltpu.async_copy(ub_hbm.at[uidx_v], ubias_v, sem_ub)
    cib = pltpu.async_copy(ib_hbm.at[iidx_v], ibias_v, sem_ib)
    cub.wait()
    cib.wait()
    cu.wait()
    ci.wait()

    lane = lax.iota(jnp.int32, L)

    def group(g, carry):
        e0 = g * L
        acc = jnp.zeros((L,), jnp.float32)
        for j in range(L):
            e = e0 + j
            p = (urows_v[e, pl.ds(0, L)] * irows_v[e, pl.ds(0, L)]
                 + urows_v[e, pl.ds(L, L)] * irows_v[e, pl.ds(L, L)]) \
                + (urows_v[e, pl.ds(2 * L, L)] * irows_v[e, pl.ds(2 * L, L)]
                   + urows_v[e, pl.ds(3 * L, L)] * irows_v[e, pl.ds(3 * L, L)])
            acc = jnp.where(lane == j, jnp.sum(p), acc)
        res_v[pl.ds(e0, L)] = (acc + ubias_v[pl.ds(e0, L)]
                               + ibias_v[pl.ds(e0, L)])
        return carry

    lax.fori_loop(0, GROUPS, group, 0)

    pltpu.sync_copy(res_v, out_hbm.at[pl.ds(base, BPW)])


def kernel(users, items, user_embeddings, item_embeddings, user_biases,
           item_biases):
    preds = _sc_gather_dot(users, items, user_embeddings, item_embeddings,
                           user_biases.reshape(-1), item_biases.reshape(-1))
    return preds.reshape(B, 1)


# speedup vs baseline: 1.0011x; 1.0011x over previous
import functools

import jax
import jax.numpy as jnp
from jax import lax
from jax.experimental import pallas as pl
from jax.experimental.pallas import tpu as pltpu
from jax.experimental.pallas import tpu_sc as plsc

B = 16384
F = 64
NC = 2
NS = 16
L = 16
NW = NC * NS
BPW = B // NW
GROUPS = BPW // L
FCH = F // L

_mesh = plsc.VectorSubcoreMesh(core_axis_name="c", subcore_axis_name="s")


@functools.partial(
    pl.kernel,
    mesh=_mesh,
    out_type=jax.ShapeDtypeStruct((B,), jnp.float32),
    compiler_params=pltpu.CompilerParams(
        needs_layout_passes=False, use_tc_tiling_on_sc=False),
    scratch_types=[
        pltpu.VMEM((BPW,), jnp.int32),
        pltpu.VMEM((BPW,), jnp.int32),
        pltpu.VMEM((BPW, F), jnp.float32),
        pltpu.VMEM((BPW, F), jnp.float32),
        pltpu.VMEM((BPW,), jnp.float32),
        pltpu.VMEM((BPW,), jnp.float32),
        pltpu.VMEM((BPW,), jnp.float32),
        pltpu.SemaphoreType.DMA,
        pltpu.SemaphoreType.DMA,
        pltpu.SemaphoreType.DMA,
        pltpu.SemaphoreType.DMA,
    ],
)
def _sc_fm(users_hbm, items_hbm, ue_hbm, ie_hbm, ub_hbm, ib_hbm, out_hbm,
           uidx_v, iidx_v, urows_v, irows_v, ubias_v, ibias_v, res_v,
           sem_u, sem_i, sem_ub, sem_ib):
    wid = lax.axis_index("s") * NC + lax.axis_index("c")
    base = wid * BPW

    pltpu.sync_copy(users_hbm.at[pl.ds(base, BPW)], uidx_v)
    pltpu.sync_copy(items_hbm.at[pl.ds(base, BPW)], iidx_v)

    cu = pltpu.async_copy(ue_hbm.at[uidx_v], urows_v, sem_u)
    ci = pltpu.async_copy(ie_hbm.at[iidx_v], irows_v, sem_i)
    cub = pltpu.async_copy(ub_hbm.at[uidx_v], ubias_v, sem_ub)
    cib = pltpu.async_copy(ib_hbm.at[iidx_v], ibias_v, sem_ib)
    cu.wait()
    ci.wait()
    cub.wait()
    cib.wait()

    lane = lax.iota(jnp.int32, L)

    def group(g, carry):
        def ex(j, acc):
            e = g * L + j
            p = jnp.zeros((L,), jnp.float32)
            for f in range(FCH):
                u = urows_v[e, pl.ds(f * L, L)]
                w = irows_v[e, pl.ds(f * L, L)]
                p = p + u * w
            s = jnp.sum(p)
            return jnp.where(lane == j, s, acc)

        acc = lax.fori_loop(0, L, ex, jnp.zeros((L,), jnp.float32))
        sl = pl.ds(g * L, L)
        res_v[sl] = acc + ubias_v[sl] + ibias_v[sl]
        return carry

    lax.fori_loop(0, GROUPS, group, 0)

    pltpu.sync_copy(res_v, out_hbm.at[pl.ds(base, BPW)])


def kernel(users, items, user_embeddings, item_embeddings, user_biases,
           item_biases):
    preds = _sc_fm(users, items, user_embeddings, item_embeddings,
                   user_biases.reshape(-1), item_biases.reshape(-1))
    return preds.reshape(B, 1)
